# baseline (device time: 83898 ns/iter reference)
import jax
import jax.numpy as jnp
from jax import lax
from jax.experimental import pallas as pl
from jax.experimental.pallas import tpu as pltpu

N_DEV = 4
S = 8


def kernel(x, w_mat, scale_x, scale_w):
    m_per, k = x.shape
    n_per = w_mat.shape[1]
    half = m_per // 2
    P = half // S

    f8 = jnp.float8_e4m3fn

    def body(x_hbm, w_hbm, sx_ref, sw_ref, out_hbm,
             xs_ref, x8_ref, wf_ref, w8_ref, ov_ref,
             copy_sems, w_sem, out_sems, cw_send, cw_recv, ccw_send, ccw_recv):
        my = lax.axis_index("i")
        left = (my + N_DEV - 1) % N_DEV
        right = (my + 1) % N_DEV

        order = [(d, s) for s in range(S) for d in (0, 1)]

        def issue_copy(i):
            d, s = order[i]
            c = pltpu.make_async_copy(
                x_hbm.at[pl.ds(d * half + s * P, P), :],
                xs_ref.at[i % 2],
                copy_sems.at[i % 2],
            )
            c.start()
            return c

        x_copies = {0: issue_copy(0), 1: issue_copy(1)}
        w_copy = pltpu.make_async_copy(w_hbm, wf_ref, w_sem)
        w_copy.start()

        barrier_sem = pltpu.get_barrier_semaphore()
        for nbr in (left, right):
            pl.semaphore_signal(
                barrier_sem, inc=1,
                device_id=(nbr,), device_id_type=pl.DeviceIdType.MESH,
            )
        pl.semaphore_wait(barrier_sem, 2)

        def cw_rows(h, s):
            o = (my + N_DEV - h) % N_DEV
            return o * m_per + s * P

        def ccw_rows(h, s):
            o = (my + h) % N_DEV
            return o * m_per + half + s * P

        def cw_send_piece(h, s):
            r = cw_rows(h, s)
            rdma = pltpu.make_async_remote_copy(
                src_ref=x8_ref.at[pl.ds(r, P), :],
                dst_ref=x8_ref.at[pl.ds(r, P), :],
                send_sem=cw_send.at[h, s],
                recv_sem=cw_recv.at[h, s],
                device_id=(right,),
                device_id_type=pl.DeviceIdType.MESH,
            )
            rdma.start()
            return rdma

        def ccw_send_piece(h, s):
            r = ccw_rows(h, s)
            rdma = pltpu.make_async_remote_copy(
                src_ref=x8_ref.at[pl.ds(r, P), :],
                dst_ref=x8_ref.at[pl.ds(r, P), :],
                send_sem=ccw_send.at[h, s],
                recv_sem=ccw_recv.at[h, s],
                device_id=(left,),
                device_id_type=pl.DeviceIdType.MESH,
            )
            rdma.start()
            return rdma

        sends = []

        for i, (d, s) in enumerate(order):
            x_copies[i].wait()
            r = cw_rows(0, s) if d == 0 else ccw_rows(0, s)
            x8_ref[pl.ds(r, P), :] = xs_ref[i % 2].astype(f8)
            sends.append(cw_send_piece(0, s) if d == 0 else ccw_send_piece(0, s))
            if i + 2 < len(order):
                x_copies[i + 2] = issue_copy(i + 2)

        scale = sx_ref[0] * sw_ref[0]

        out_state = {"n": 0, 0: None, 1: None}

        def store(row0):
            slot = out_state["n"] % 2
            if out_state[slot] is not None:
                out_state[slot].wait()
            chunk = x8_ref[pl.ds(row0, half), :]
            acc = jnp.dot(chunk, w8_ref[...], preferred_element_type=jnp.float32)
            y = acc * scale
            ov_ref[slot] = y / (1.0 + jnp.exp(-y))
            c = pltpu.make_async_copy(
                ov_ref.at[slot], out_hbm.at[pl.ds(row0, half), :],
                out_sems.at[slot],
            )
            c.start()
            out_state[slot] = c
            out_state["n"] += 1

        w_copy.wait()
        w8_ref[...] = wf_ref[...].astype(f8)
        store(my * m_per)
        store(my * m_per + half)

        def compute_gen(h):
            store(cw_rows(h + 1, 0))
            store(ccw_rows(h + 1, 0))

        for h in range(1, N_DEV - 1):
            for s in range(S):
                pltpu.make_async_copy(
                    x8_ref.at[pl.ds(0, P), :], x8_ref.at[pl.ds(0, P), :],
                    cw_recv.at[h - 1, s],
                ).wait()
                sends.append(cw_send_piece(h, s))
                pltpu.make_async_copy(
                    x8_ref.at[pl.ds(0, P), :], x8_ref.at[pl.ds(0, P), :],
                    ccw_recv.at[h - 1, s],
                ).wait()
                sends.append(ccw_send_piece(h, s))
            compute_gen(h - 1)

        for s in range(S):
            pltpu.make_async_copy(
                x8_ref.at[pl.ds(0, P), :], x8_ref.at[pl.ds(0, P), :],
                cw_recv.at[N_DEV - 2, s],
            ).wait()
            pltpu.make_async_copy(
                x8_ref.at[pl.ds(0, P), :], x8_ref.at[pl.ds(0, P), :],
                ccw_recv.at[N_DEV - 2, s],
            ).wait()
        compute_gen(N_DEV - 2)

        for slot in (0, 1):
            if out_state[slot] is not None:
                out_state[slot].wait()
        for rdma in sends:
            rdma.wait_send()

    return pl.pallas_call(
        body,
        out_shape=jax.ShapeDtypeStruct((N_DEV * m_per, n_per), jnp.float32),
        in_specs=[
            pl.BlockSpec(memory_space=pl.ANY),
            pl.BlockSpec(memory_space=pl.ANY),
            pl.BlockSpec(memory_space=pltpu.SMEM),
            pl.BlockSpec(memory_space=pltpu.SMEM),
        ],
        out_specs=pl.BlockSpec(memory_space=pl.ANY),
        scratch_shapes=[
            pltpu.VMEM((2, P, k), jnp.float32),
            pltpu.VMEM((N_DEV * m_per, k), f8),
            pltpu.VMEM((k, n_per), jnp.float32),
            pltpu.VMEM((k, n_per), f8),
            pltpu.VMEM((2, half, n_per), jnp.float32),
            pltpu.SemaphoreType.DMA((2,)),
            pltpu.SemaphoreType.DMA,
            pltpu.SemaphoreType.DMA((2,)),
            pltpu.SemaphoreType.DMA((N_DEV - 1, S)),
            pltpu.SemaphoreType.DMA((N_DEV - 1, S)),
            pltpu.SemaphoreType.DMA((N_DEV - 1, S)),
            pltpu.SemaphoreType.DMA((N_DEV - 1, S)),
        ],
        compiler_params=pltpu.CompilerParams(collective_id=0),
    )(x, w_mat, scale_x, scale_w)


# device time: 80419 ns/iter; 1.0433x vs baseline; 1.0433x over previous
import jax
import jax.numpy as jnp
from jax import lax
from jax.experimental import pallas as pl
from jax.experimental.pallas import tpu as pltpu

N_DEV = 4
S = 8


def kernel(x, w_mat, scale_x, scale_w):
    m_per, k = x.shape
    n_per = w_mat.shape[1]
    half = m_per // 2
    P = half // S

    f8 = jnp.float8_e4m3fn

    def body(x_hbm, w_hbm, sx_ref, sw_ref, out_hbm,
             xs_ref, x8_ref, wf_ref, w8_ref, ov_ref,
             copy_sems, w_sem, out_sems, cw_send, cw_recv, ccw_send, ccw_recv):
        my = lax.axis_index("i")
        left = (my + N_DEV - 1) % N_DEV
        right = (my + 1) % N_DEV

        order = [(d, s) for s in range(S) for d in (0, 1)]

        def issue_copy(i):
            d, s = order[i]
            c = pltpu.make_async_copy(
                x_hbm.at[pl.ds(d * half + s * P, P), :],
                xs_ref.at[i % 2],
                copy_sems.at[i % 2],
            )
            c.start()
            return c

        x_copies = {0: issue_copy(0), 1: issue_copy(1)}
        w_copy = pltpu.make_async_copy(w_hbm, wf_ref, w_sem)
        w_copy.start()

        barrier_sem = pltpu.get_barrier_semaphore()
        for nbr in (left, right):
            pl.semaphore_signal(
                barrier_sem, inc=1,
                device_id=(nbr,), device_id_type=pl.DeviceIdType.MESH,
            )
        pl.semaphore_wait(barrier_sem, 2)

        def cw_rows(h, s):
            o = (my + N_DEV - h) % N_DEV
            return o * m_per + s * P

        def ccw_rows(h, s):
            o = (my + h) % N_DEV
            return o * m_per + half + s * P

        def cw_send_piece(h, s):
            r = cw_rows(h, s)
            rdma = pltpu.make_async_remote_copy(
                src_ref=x8_ref.at[pl.ds(r, P), :],
                dst_ref=x8_ref.at[pl.ds(r, P), :],
                send_sem=cw_send.at[h, s],
                recv_sem=cw_recv.at[h, s],
                device_id=(right,),
                device_id_type=pl.DeviceIdType.MESH,
            )
            rdma.start()
            return rdma

        def ccw_send_piece(h, s):
            r = ccw_rows(h, s)
            rdma = pltpu.make_async_remote_copy(
                src_ref=x8_ref.at[pl.ds(r, P), :],
                dst_ref=x8_ref.at[pl.ds(r, P), :],
                send_sem=ccw_send.at[h, s],
                recv_sem=ccw_recv.at[h, s],
                device_id=(left,),
                device_id_type=pl.DeviceIdType.MESH,
            )
            rdma.start()
            return rdma

        sends = []

        for i, (d, s) in enumerate(order):
            x_copies[i].wait()
            r = cw_rows(0, s) if d == 0 else ccw_rows(0, s)
            x8_ref[pl.ds(r, P), :] = xs_ref[i % 2].astype(f8)
            sends.append(cw_send_piece(0, s) if d == 0 else ccw_send_piece(0, s))
            if i + 2 < len(order):
                x_copies[i + 2] = issue_copy(i + 2)

        scale = sx_ref[0] * sw_ref[0]

        out_state = {"n": 0, 0: None, 1: None}

        def store(row0):
            slot = out_state["n"] % 2
            if out_state[slot] is not None:
                out_state[slot].wait()
            chunk = x8_ref[pl.ds(row0, half), :]
            acc = jnp.dot(chunk, w8_ref[...], preferred_element_type=jnp.float32)
            y = acc * scale
            ov_ref[slot] = y / (1.0 + jnp.exp(-y))
            c = pltpu.make_async_copy(
                ov_ref.at[slot], out_hbm.at[pl.ds(row0, half), :],
                out_sems.at[slot],
            )
            c.start()
            out_state[slot] = c
            out_state["n"] += 1

        w_copy.wait()
        w8_ref[...] = wf_ref[...].astype(f8)
        store(my * m_per)
        store(my * m_per + half)

        def compute_gen(h):
            store(cw_rows(h + 1, 0))
            store(ccw_rows(h + 1, 0))

        for h in range(1, N_DEV - 1):
            for s in range(S):
                pltpu.make_async_copy(
                    x8_ref.at[pl.ds(0, P), :], x8_ref.at[pl.ds(0, P), :],
                    cw_recv.at[h - 1, s],
                ).wait()
                sends.append(cw_send_piece(h, s))
                pltpu.make_async_copy(
                    x8_ref.at[pl.ds(0, P), :], x8_ref.at[pl.ds(0, P), :],
                    ccw_recv.at[h - 1, s],
                ).wait()
                sends.append(ccw_send_piece(h, s))
            compute_gen(h - 1)

        for s in range(S):
            pltpu.make_async_copy(
                x8_ref.at[pl.ds(0, P), :], x8_ref.at[pl.ds(0, P), :],
                cw_recv.at[N_DEV - 2, s],
            ).wait()
            pltpu.make_async_copy(
                x8_ref.at[pl.ds(0, P), :], x8_ref.at[pl.ds(0, P), :],
                ccw_recv.at[N_DEV - 2, s],
            ).wait()
        compute_gen(N_DEV - 2)

        for slot in (0, 1):
            if out_state[slot] is not None:
                out_state[slot].wait()
        for rdma in sends:
            rdma.wait_send()

    return pl.pallas_call(
        body,
        out_shape=jax.ShapeDtypeStruct((N_DEV * m_per, n_per), jnp.float32),
        in_specs=[
            pl.BlockSpec(memory_space=pl.ANY),
            pl.BlockSpec(memory_space=pl.ANY),
            pl.BlockSpec(memory_space=pltpu.SMEM),
            pl.BlockSpec(memory_space=pltpu.SMEM),
        ],
        out_specs=pl.BlockSpec(memory_space=pltpu.MemorySpace.HBM),
        scratch_shapes=[
            pltpu.VMEM((2, P, k), jnp.float32),
            pltpu.VMEM((N_DEV * m_per, k), f8),
            pltpu.VMEM((k, n_per), jnp.float32),
            pltpu.VMEM((k, n_per), f8),
            pltpu.VMEM((2, half, n_per), jnp.float32),
            pltpu.SemaphoreType.DMA((2,)),
            pltpu.SemaphoreType.DMA,
            pltpu.SemaphoreType.DMA((2,)),
            pltpu.SemaphoreType.DMA((N_DEV - 1, S)),
            pltpu.SemaphoreType.DMA((N_DEV - 1, S)),
            pltpu.SemaphoreType.DMA((N_DEV - 1, S)),
            pltpu.SemaphoreType.DMA((N_DEV - 1, S)),
        ],
        compiler_params=pltpu.CompilerParams(collective_id=0),
    )(
        pltpu.with_memory_space_constraint(x, pltpu.MemorySpace.HBM),
        pltpu.with_memory_space_constraint(w_mat, pltpu.MemorySpace.HBM),
        scale_x,
        scale_w,
    )


# device time: 78922 ns/iter; 1.0630x vs baseline; 1.0190x over previous
import jax
import jax.numpy as jnp
from jax import lax
from jax.experimental import pallas as pl
from jax.experimental.pallas import tpu as pltpu

N_DEV = 4
S = 8


def kernel(x, w_mat, scale_x, scale_w):
    m_per, k = x.shape
    n_per = w_mat.shape[1]
    half = m_per // 2
    P = half // S

    f8 = jnp.float8_e4m3fn

    def body(x_hbm, w_hbm, sx_ref, sw_ref, out_hbm,
             xs_ref, x8_ref, wf_ref, w8_ref, ov_ref,
             copy_sems, w_sem, out_sems, cw_send, cw_recv, ccw_send, ccw_recv):
        my = lax.axis_index("i")
        left = (my + N_DEV - 1) % N_DEV
        right = (my + 1) % N_DEV

        order = [(d, s) for s in range(S) for d in (0, 1)]

        def issue_copy(i):
            d, s = order[i]
            c = pltpu.make_async_copy(
                x_hbm.at[pl.ds(d * half + s * P, P), :],
                xs_ref.at[i % 2],
                copy_sems.at[i % 2],
            )
            c.start()
            return c

        x_copies = {0: issue_copy(0), 1: issue_copy(1)}
        w_copy = pltpu.make_async_copy(w_hbm, wf_ref, w_sem)
        w_copy.start()

        barrier_sem = pltpu.get_barrier_semaphore()
        for nbr in (left, right):
            pl.semaphore_signal(
                barrier_sem, inc=1,
                device_id=(nbr,), device_id_type=pl.DeviceIdType.MESH,
            )
        pl.semaphore_wait(barrier_sem, 2)

        def cw_rows(h, s):
            o = (my + N_DEV - h) % N_DEV
            return o * m_per + s * P

        def ccw_rows(h, s):
            o = (my + h) % N_DEV
            return o * m_per + half + s * P

        def cw_send_piece(h, s):
            r = cw_rows(h, s)
            rdma = pltpu.make_async_remote_copy(
                src_ref=x8_ref.at[pl.ds(r, P), :],
                dst_ref=x8_ref.at[pl.ds(r, P), :],
                send_sem=cw_send.at[h, s],
                recv_sem=cw_recv.at[h, s],
                device_id=(right,),
                device_id_type=pl.DeviceIdType.MESH,
            )
            rdma.start()
            return rdma

        def ccw_send_piece(h, s):
            r = ccw_rows(h, s)
            rdma = pltpu.make_async_remote_copy(
                src_ref=x8_ref.at[pl.ds(r, P), :],
                dst_ref=x8_ref.at[pl.ds(r, P), :],
                send_sem=ccw_send.at[h, s],
                recv_sem=ccw_recv.at[h, s],
                device_id=(left,),
                device_id_type=pl.DeviceIdType.MESH,
            )
            rdma.start()
            return rdma

        sends = []

        for i, (d, s) in enumerate(order):
            x_copies[i].wait()
            r = cw_rows(0, s) if d == 0 else ccw_rows(0, s)
            x8_ref[pl.ds(r, P), :] = xs_ref[i % 2].astype(f8)
            sends.append(cw_send_piece(0, s) if d == 0 else ccw_send_piece(0, s))
            if i + 2 < len(order):
                x_copies[i + 2] = issue_copy(i + 2)

        scale = sx_ref[0] * sw_ref[0]

        out_state = {"n": 0, 0: None, 1: None}

        def store(row0, height=half):
            slot = out_state["n"] % 2
            if out_state[slot] is not None:
                out_state[slot].wait()
            chunk = x8_ref[pl.ds(row0, height), :]
            acc = jnp.dot(chunk, w8_ref[...], preferred_element_type=jnp.float32)
            y = acc * scale
            ov_ref[slot, pl.ds(0, height), :] = y / (1.0 + jnp.exp(-y))
            c = pltpu.make_async_copy(
                ov_ref.at[slot, pl.ds(0, height), :],
                out_hbm.at[pl.ds(row0, height), :],
                out_sems.at[slot],
            )
            c.start()
            out_state[slot] = c
            out_state["n"] += 1

        w_copy.wait()
        w8_ref[...] = wf_ref[...].astype(f8)
        store(my * m_per)
        store(my * m_per + half)

        def compute_gen(h):
            store(cw_rows(h + 1, 0))
            store(ccw_rows(h + 1, 0))

        for h in range(1, N_DEV - 1):
            for s in range(S):
                pltpu.make_async_copy(
                    x8_ref.at[pl.ds(0, P), :], x8_ref.at[pl.ds(0, P), :],
                    cw_recv.at[h - 1, s],
                ).wait()
                sends.append(cw_send_piece(h, s))
                pltpu.make_async_copy(
                    x8_ref.at[pl.ds(0, P), :], x8_ref.at[pl.ds(0, P), :],
                    ccw_recv.at[h - 1, s],
                ).wait()
                sends.append(ccw_send_piece(h, s))
            compute_gen(h - 1)

        G = S // 2
        for g in range(2):
            for s in range(g * G, (g + 1) * G):
                pltpu.make_async_copy(
                    x8_ref.at[pl.ds(0, P), :], x8_ref.at[pl.ds(0, P), :],
                    cw_recv.at[N_DEV - 2, s],
                ).wait()
            store(cw_rows(N_DEV - 1, g * G), G * P)
            for s in range(g * G, (g + 1) * G):
                pltpu.make_async_copy(
                    x8_ref.at[pl.ds(0, P), :], x8_ref.at[pl.ds(0, P), :],
                    ccw_recv.at[N_DEV - 2, s],
                ).wait()
            store(ccw_rows(N_DEV - 1, g * G), G * P)

        for slot in (0, 1):
            if out_state[slot] is not None:
                out_state[slot].wait()
        for rdma in sends:
            rdma.wait_send()

    return pl.pallas_call(
        body,
        out_shape=jax.ShapeDtypeStruct((N_DEV * m_per, n_per), jnp.float32),
        in_specs=[
            pl.BlockSpec(memory_space=pl.ANY),
            pl.BlockSpec(memory_space=pl.ANY),
            pl.BlockSpec(memory_space=pltpu.SMEM),
            pl.BlockSpec(memory_space=pltpu.SMEM),
        ],
        out_specs=pl.BlockSpec(memory_space=pltpu.MemorySpace.HBM),
        scratch_shapes=[
            pltpu.VMEM((2, P, k), jnp.float32),
            pltpu.VMEM((N_DEV * m_per, k), f8),
            pltpu.VMEM((k, n_per), jnp.float32),
            pltpu.VMEM((k, n_per), f8),
            pltpu.VMEM((2, half, n_per), jnp.float32),
            pltpu.SemaphoreType.DMA((2,)),
            pltpu.SemaphoreType.DMA,
            pltpu.SemaphoreType.DMA((2,)),
            pltpu.SemaphoreType.DMA((N_DEV - 1, S)),
            pltpu.SemaphoreType.DMA((N_DEV - 1, S)),
            pltpu.SemaphoreType.DMA((N_DEV - 1, S)),
            pltpu.SemaphoreType.DMA((N_DEV - 1, S)),
        ],
        compiler_params=pltpu.CompilerParams(collective_id=0),
    )(
        pltpu.with_memory_space_constraint(x, pltpu.MemorySpace.HBM),
        pltpu.with_memory_space_constraint(w_mat, pltpu.MemorySpace.HBM),
        scale_x,
        scale_w,
    )
